# baseline (speedup 1.0000x reference)
import functools

import jax
import jax.numpy as jnp
from jax import lax
from jax.experimental import pallas as pl
from jax.experimental.pallas import tpu as pltpu
from jax.experimental.pallas import tpu_sc as plsc

_NC = 2
_NS = 16
_L = 16
_NW = _NC * _NS


def _make_sc_kernel(total):
    per_w = total // _NW
    nvec = per_w // _L
    mesh = plsc.VectorSubcoreMesh(core_axis_name="c", subcore_axis_name="s")

    @functools.partial(
        pl.kernel,
        out_type=jax.ShapeDtypeStruct((total,), jnp.float32),
        mesh=mesh,
        scratch_types=[pltpu.VMEM((per_w,), jnp.float32)],
    )
    def k(x_hbm, o_hbm, buf):
        wid = lax.axis_index("s") * _NC + lax.axis_index("c")
        base = wid * per_w
        pltpu.sync_copy(x_hbm.at[pl.ds(base, per_w)], buf)
        for i in range(nvec):
            sl = pl.ds(i * _L, _L)
            buf[sl] = buf[sl] * 32.0 + 64.0
        pltpu.sync_copy(buf, o_hbm.at[pl.ds(base, per_w)])

    return k


def kernel(input_xyzs):
    xt = input_xyzs.T
    flat = xt.reshape(-1)
    out = _make_sc_kernel(flat.shape[0])(flat)
    return out.reshape(xt.shape).T


# speedup vs baseline: 14.3519x; 14.3519x over previous
import jax
import jax.numpy as jnp
from jax.experimental import pallas as pl


def _ew_kernel(x_ref, o_ref):
    o_ref[...] = x_ref[...] * 32.0 + 64.0


def kernel(input_xyzs):
    xt = input_xyzs.T
    out = pl.pallas_call(
        _ew_kernel,
        out_shape=jax.ShapeDtypeStruct(xt.shape, xt.dtype),
    )(xt)
    return out.T
